# baseline (speedup 1.0000x reference)
import functools

import jax
import jax.numpy as jnp
from jax import lax
from jax.experimental import pallas as pl
from jax.experimental.pallas import tpu as pltpu
from jax.experimental.pallas import tpu_sc as plsc

N_NODES = 10000
NPAD = 10240
N_EDGES = 320000
EPAD = 327680
CHUNK = 128
CHUNKS_PER_TILE = 80
PCHUNK = 64
NBUF = 4
PBLK = 16
CHUNKS_C0 = 256
CHUNKS_C1 = 64
ROWS_PER_TILE = NPAD // 16
HID = 64
IN_CH = 128
DEGW = 16

_MESH = plsc.VectorSubcoreMesh(core_axis_name="c", subcore_axis_name="s",
                               num_cores=2, num_subcores=16)


@functools.partial(
    pl.kernel,
    out_type=[jax.ShapeDtypeStruct((2, NPAD, DEGW), jnp.float32),
              jax.ShapeDtypeStruct((2, NPAD, DEGW), jnp.float32)],
    mesh=_MESH,
    scratch_types=[
        pltpu.VMEM((CHUNKS_PER_TILE, CHUNK), jnp.int32),
        pltpu.VMEM((CHUNKS_PER_TILE, CHUNK), jnp.int32),
        pltpu.VMEM((CHUNK, DEGW), jnp.float32),
        pltpu.VMEM((CHUNK, DEGW), jnp.float32),
        pltpu.VMEM_SHARED((NPAD, DEGW), jnp.float32),
        pltpu.VMEM_SHARED((NPAD, DEGW), jnp.float32),
    ],
    compiler_params=pltpu.CompilerParams(use_tc_tiling_on_sc=False),
)
def _sc_degrees(src_hbm, dst_hbm, deg0_out, deg1_out,
                src_v, dst_v, z_v, one_v, acc0, acc1):
    c = lax.axis_index("c")
    s = lax.axis_index("s")
    wid = c * 16 + s

    def fill(i, _):
        for k in range(DEGW // 16):
            z_v[i, pl.ds(k * 16, 16)] = jnp.zeros((16,), jnp.float32)
            one_v[i, pl.ds(k * 16, 16)] = jnp.ones((16,), jnp.float32)
        return 0

    lax.fori_loop(0, CHUNK, fill, 0)
    for k in range(ROWS_PER_TILE // CHUNK):
        off = s * ROWS_PER_TILE + k * CHUNK
        pltpu.sync_copy(z_v, acc0.at[pl.ds(off, CHUNK)])
        pltpu.sync_copy(z_v, acc1.at[pl.ds(off, CHUNK)])
    plsc.subcore_barrier()

    pltpu.sync_copy(src_hbm.at[pl.ds(wid * CHUNKS_PER_TILE, CHUNKS_PER_TILE)], src_v)
    pltpu.sync_copy(dst_hbm.at[pl.ds(wid * CHUNKS_PER_TILE, CHUNKS_PER_TILE)], dst_v)

    def body(j, _):
        pltpu.sync_copy(one_v, acc0.at[dst_v.at[j]], add=True)
        pltpu.sync_copy(one_v, acc1.at[src_v.at[j]], add=True)
        return 0

    lax.fori_loop(0, CHUNKS_PER_TILE, body, 0)
    plsc.subcore_barrier()
    pltpu.sync_copy(acc0.at[pl.ds(s * ROWS_PER_TILE, ROWS_PER_TILE)],
                    deg0_out.at[c, pl.ds(s * ROWS_PER_TILE, ROWS_PER_TILE)])
    pltpu.sync_copy(acc1.at[pl.ds(s * ROWS_PER_TILE, ROWS_PER_TILE)],
                    deg1_out.at[c, pl.ds(s * ROWS_PER_TILE, ROWS_PER_TILE)])


@functools.partial(
    pl.kernel,
    out_type=[jax.ShapeDtypeStruct((2, NPAD, HID), jnp.float32),
              jax.ShapeDtypeStruct((2, NPAD, HID), jnp.float32)],
    mesh=_MESH,
    scratch_types=(
        [pltpu.VMEM((PBLK, PCHUNK), jnp.int32),
         pltpu.VMEM((PBLK, PCHUNK), jnp.int32)]
        + [pltpu.VMEM((PCHUNK, HID), jnp.float32) for _ in range(2 * NBUF)]
        + [pltpu.SemaphoreType.DMA for _ in range(2 * NBUF + 2)]
        + [pltpu.VMEM_SHARED((NPAD, HID), jnp.float32),
           pltpu.VMEM_SHARED((NPAD, HID), jnp.float32)]
    ),
    compiler_params=pltpu.CompilerParams(use_tc_tiling_on_sc=False),
)
def _sc_propagate(u0_hbm, u1_hbm, src_hbm, dst_hbm, s0_out, s1_out,
                  src_v, dst_v, *rest):
    r0 = rest[0:NBUF]
    r1 = rest[NBUF:2 * NBUF]
    g0 = rest[2 * NBUF:3 * NBUF]
    g1 = rest[3 * NBUF:4 * NBUF]
    s0sem = rest[4 * NBUF]
    s1sem = rest[4 * NBUF + 1]
    acc0 = rest[4 * NBUF + 2]
    acc1 = rest[4 * NBUF + 3]
    c = lax.axis_index("c")
    s = lax.axis_index("s")

    def fill(i, _):
        for k in range(HID // 16):
            r0[0][i, pl.ds(k * 16, 16)] = jnp.zeros((16,), jnp.float32)
        return 0

    lax.fori_loop(0, PCHUNK, fill, 0)
    for k in range(ROWS_PER_TILE // PCHUNK):
        off = s * ROWS_PER_TILE + k * PCHUNK
        pltpu.async_copy(r0[0], acc0.at[pl.ds(off, PCHUNK)], g0[0])
        pltpu.async_copy(r0[0], acc1.at[pl.ds(off, PCHUNK)], g0[0])
    for k in range(ROWS_PER_TILE // PCHUNK):
        off = s * ROWS_PER_TILE + k * PCHUNK
        pltpu.make_async_copy(r0[0], acc0.at[pl.ds(off, PCHUNK)], g0[0]).wait()
        pltpu.make_async_copy(r0[0], acc1.at[pl.ds(off, PCHUNK)], g0[0]).wait()
    plsc.subcore_barrier()

    def run(tile_base, n_chunks):
        for ib in range(n_chunks // PBLK):
            base = tile_base + ib * PBLK
            pltpu.sync_copy(src_hbm.at[pl.ds(base, PBLK)], src_v)
            pltpu.sync_copy(dst_hbm.at[pl.ds(base, PBLK)], dst_v)

            for p in range(NBUF - 1):
                pltpu.async_copy(u0_hbm.at[src_v.at[p]], r0[p], g0[p])
                pltpu.async_copy(u1_hbm.at[dst_v.at[p]], r1[p], g1[p])

            def block(bi, _):
                for t in range(NBUF):
                    j = bi * NBUF + t
                    nf = (t + NBUF - 1) % NBUF

                    @pl.when(j >= 1)
                    def _():
                        pltpu.make_async_copy(
                            r0[nf], acc0.at[dst_v.at[j - 1]], s0sem).wait()
                        pltpu.make_async_copy(
                            r1[nf], acc1.at[src_v.at[j - 1]], s1sem).wait()

                    @pl.when(j + NBUF - 1 < PBLK)
                    def _():
                        pltpu.async_copy(
                            u0_hbm.at[src_v.at[j + NBUF - 1]], r0[nf], g0[nf])
                        pltpu.async_copy(
                            u1_hbm.at[dst_v.at[j + NBUF - 1]], r1[nf], g1[nf])

                    pltpu.make_async_copy(u0_hbm.at[src_v.at[j]], r0[t], g0[t]).wait()
                    pltpu.async_copy(r0[t], acc0.at[dst_v.at[j]], s0sem, add=True)
                    pltpu.make_async_copy(u1_hbm.at[dst_v.at[j]], r1[t], g1[t]).wait()
                    pltpu.async_copy(r1[t], acc1.at[src_v.at[j]], s1sem, add=True)
                return 0

            lax.fori_loop(0, PBLK // NBUF, block, 0)
            last = PBLK - 1
            lt = last % NBUF
            pltpu.make_async_copy(r0[lt], acc0.at[dst_v.at[last]], s0sem).wait()
            pltpu.make_async_copy(r1[lt], acc1.at[src_v.at[last]], s1sem).wait()

    @pl.when(c == 0)
    def _():
        run(s * CHUNKS_C0, CHUNKS_C0)

    @pl.when(c == 1)
    def _():
        run(16 * CHUNKS_C0 + s * CHUNKS_C1, CHUNKS_C1)

    plsc.subcore_barrier()
    pltpu.async_copy(acc0.at[pl.ds(s * ROWS_PER_TILE, ROWS_PER_TILE)],
                     s0_out.at[c, pl.ds(s * ROWS_PER_TILE, ROWS_PER_TILE)], g0[0])
    pltpu.async_copy(acc1.at[pl.ds(s * ROWS_PER_TILE, ROWS_PER_TILE)],
                     s1_out.at[c, pl.ds(s * ROWS_PER_TILE, ROWS_PER_TILE)], g0[1])
    pltpu.make_async_copy(acc0.at[pl.ds(s * ROWS_PER_TILE, ROWS_PER_TILE)],
                          s0_out.at[c, pl.ds(s * ROWS_PER_TILE, ROWS_PER_TILE)], g0[0]).wait()
    pltpu.make_async_copy(acc1.at[pl.ds(s * ROWS_PER_TILE, ROWS_PER_TILE)],
                          s1_out.at[c, pl.ds(s * ROWS_PER_TILE, ROWS_PER_TILE)], g0[1]).wait()


_BLK = 256
_GRID = NPAD // _BLK


def _tc_prep_body(x_ref, d0_ref, d1_ref, w00_ref, w10_ref,
                  u0_ref, u1_ref, di0_ref, di1_ref):
    deg0 = d0_ref[0, :, 0] + d0_ref[1, :, 0] + 1.0
    deg1 = d1_ref[0, :, 0] + d1_ref[1, :, 0] + 1.0
    dinv0 = lax.rsqrt(deg0)
    dinv1 = lax.rsqrt(deg1)
    x = x_ref[...]
    u0_ref[...] = jnp.dot(x, w00_ref[...], preferred_element_type=jnp.float32) * dinv0[:, None]
    u1_ref[...] = jnp.dot(x, w10_ref[...], preferred_element_type=jnp.float32) * dinv1[:, None]
    di0_ref[...] = dinv0
    di1_ref[...] = dinv1


def _tc_prep(x_pad, deg0p, deg1p, w00, w10):
    return pl.pallas_call(
        _tc_prep_body,
        grid=(_GRID,),
        in_specs=[
            pl.BlockSpec((_BLK, IN_CH), lambda i: (i, 0)),
            pl.BlockSpec((2, _BLK, DEGW), lambda i: (0, i, 0)),
            pl.BlockSpec((2, _BLK, DEGW), lambda i: (0, i, 0)),
            pl.BlockSpec((IN_CH, HID), lambda i: (0, 0)),
            pl.BlockSpec((IN_CH, HID), lambda i: (0, 0)),
        ],
        out_specs=[
            pl.BlockSpec((_BLK, HID), lambda i: (i, 0)),
            pl.BlockSpec((_BLK, HID), lambda i: (i, 0)),
            pl.BlockSpec((_BLK,), lambda i: (i,)),
            pl.BlockSpec((_BLK,), lambda i: (i,)),
        ],
        out_shape=[
            jax.ShapeDtypeStruct((NPAD, HID), jnp.float32),
            jax.ShapeDtypeStruct((NPAD, HID), jnp.float32),
            jax.ShapeDtypeStruct((NPAD,), jnp.float32),
            jax.ShapeDtypeStruct((NPAD,), jnp.float32),
        ],
    )(x_pad, deg0p, deg1p, w00, w10)


def _tc_mid_body(s0_ref, s1_ref, u0_ref, u1_ref, di0_ref, di1_ref,
                 b00_ref, b10_ref, w01_ref, w11_ref, v0_ref, v1_ref):
    di0 = di0_ref[...][:, None]
    di1 = di1_ref[...][:, None]
    h0 = jnp.maximum(di0 * (s0_ref[0] + s0_ref[1] + u0_ref[...]) + b00_ref[...], 0.0)
    h1 = jnp.maximum(di1 * (s1_ref[0] + s1_ref[1] + u1_ref[...]) + b10_ref[...], 0.0)
    v0_ref[...] = jnp.dot(h0, w01_ref[...], preferred_element_type=jnp.float32) * di0
    v1_ref[...] = jnp.dot(h1, w11_ref[...], preferred_element_type=jnp.float32) * di1


def _tc_mid(s0p, s1p, u0, u1, di0, di1, b00, b10, w01, w11):
    return pl.pallas_call(
        _tc_mid_body,
        grid=(_GRID,),
        in_specs=[
            pl.BlockSpec((2, _BLK, HID), lambda i: (0, i, 0)),
            pl.BlockSpec((2, _BLK, HID), lambda i: (0, i, 0)),
            pl.BlockSpec((_BLK, HID), lambda i: (i, 0)),
            pl.BlockSpec((_BLK, HID), lambda i: (i, 0)),
            pl.BlockSpec((_BLK,), lambda i: (i,)),
            pl.BlockSpec((_BLK,), lambda i: (i,)),
            pl.BlockSpec((HID,), lambda i: (0,)),
            pl.BlockSpec((HID,), lambda i: (0,)),
            pl.BlockSpec((HID, HID), lambda i: (0, 0)),
            pl.BlockSpec((HID, HID), lambda i: (0, 0)),
        ],
        out_specs=[
            pl.BlockSpec((_BLK, HID), lambda i: (i, 0)),
            pl.BlockSpec((_BLK, HID), lambda i: (i, 0)),
        ],
        out_shape=[
            jax.ShapeDtypeStruct((NPAD, HID), jnp.float32),
            jax.ShapeDtypeStruct((NPAD, HID), jnp.float32),
        ],
    )(s0p, s1p, u0, u1, di0, di1, b00, b10, w01, w11)


def _tc_final_body(s0_ref, s1_ref, v0_ref, v1_ref, di0_ref, di1_ref,
                   b01_ref, b11_ref, out_ref):
    di0 = di0_ref[...][:, None]
    di1 = di1_ref[...][:, None]
    out_ref[:, :HID] = di0 * (s0_ref[0] + s0_ref[1] + v0_ref[...]) + b01_ref[...]
    out_ref[:, HID:] = di1 * (s1_ref[0] + s1_ref[1] + v1_ref[...]) + b11_ref[...]


def _tc_final(s0p, s1p, v0, v1, di0, di1, b01, b11):
    return pl.pallas_call(
        _tc_final_body,
        grid=(_GRID,),
        in_specs=[
            pl.BlockSpec((2, _BLK, HID), lambda i: (0, i, 0)),
            pl.BlockSpec((2, _BLK, HID), lambda i: (0, i, 0)),
            pl.BlockSpec((_BLK, HID), lambda i: (i, 0)),
            pl.BlockSpec((_BLK, HID), lambda i: (i, 0)),
            pl.BlockSpec((_BLK,), lambda i: (i,)),
            pl.BlockSpec((_BLK,), lambda i: (i,)),
            pl.BlockSpec((HID,), lambda i: (0,)),
            pl.BlockSpec((HID,), lambda i: (0,)),
        ],
        out_specs=pl.BlockSpec((_BLK, 2 * HID), lambda i: (i, 0)),
        out_shape=jax.ShapeDtypeStruct((NPAD, 2 * HID), jnp.float32),
    )(s0p, s1p, v0, v1, di0, di1, b01, b11)


def kernel(x, edge_index, adj_mat_ls, W0_0, b0_0, W0_1, b0_1, W1_0, b1_0, W1_1, b1_1):
    ei = edge_index.astype(jnp.int32)
    pad = jnp.full((EPAD - N_EDGES,), N_NODES, jnp.int32)
    src2d = jnp.concatenate([ei[0], pad]).reshape(EPAD // CHUNK, CHUNK)
    dst2d = jnp.concatenate([ei[1], pad]).reshape(EPAD // CHUNK, CHUNK)
    x_pad = jnp.pad(x, ((0, NPAD - N_NODES), (0, 0)))

    src64 = src2d.reshape(EPAD // PCHUNK, PCHUNK)
    dst64 = dst2d.reshape(EPAD // PCHUNK, PCHUNK)

    deg0p, deg1p = _sc_degrees(src2d, dst2d)
    u0, u1, di0, di1 = _tc_prep(x_pad, deg0p, deg1p, W0_0, W1_0)
    s0p, s1p = _sc_propagate(u0, u1, src64, dst64)
    v0, v1 = _tc_mid(s0p, s1p, u0, u1, di0, di1, b0_0, b1_0, W0_1, W1_1)
    t0p, t1p = _sc_propagate(v0, v1, src64, dst64)
    out = _tc_final(t0p, t1p, v0, v1, di0, di1, b0_1, b1_1)
    return out[:N_NODES]


# speedup vs baseline: 1.0210x; 1.0210x over previous
import functools

import jax
import jax.numpy as jnp
from jax import lax
from jax.experimental import pallas as pl
from jax.experimental.pallas import tpu as pltpu
from jax.experimental.pallas import tpu_sc as plsc

N_NODES = 10000
NPAD = 10240
N_EDGES = 320000
EPAD = 327680
CHUNK = 128
CHUNKS_PER_TILE = 80
PCHUNK = 64
NBUF = 4
PBLK = 16
CHUNKS_C0 = 240
CHUNKS_C1 = 80
ROWS_PER_TILE = NPAD // 16
HID = 64
IN_CH = 128
DEGW = 16

_MESH = plsc.VectorSubcoreMesh(core_axis_name="c", subcore_axis_name="s",
                               num_cores=2, num_subcores=16)


@functools.partial(
    pl.kernel,
    out_type=[jax.ShapeDtypeStruct((2, NPAD, DEGW), jnp.float32),
              jax.ShapeDtypeStruct((2, NPAD, DEGW), jnp.float32)],
    mesh=_MESH,
    scratch_types=[
        pltpu.VMEM((CHUNKS_PER_TILE, CHUNK), jnp.int32),
        pltpu.VMEM((CHUNKS_PER_TILE, CHUNK), jnp.int32),
        pltpu.VMEM((CHUNK, DEGW), jnp.float32),
        pltpu.VMEM((CHUNK, DEGW), jnp.float32),
        pltpu.VMEM_SHARED((NPAD, DEGW), jnp.float32),
        pltpu.VMEM_SHARED((NPAD, DEGW), jnp.float32),
    ],
    compiler_params=pltpu.CompilerParams(use_tc_tiling_on_sc=False),
)
def _sc_degrees(src_hbm, dst_hbm, deg0_out, deg1_out,
                src_v, dst_v, z_v, one_v, acc0, acc1):
    c = lax.axis_index("c")
    s = lax.axis_index("s")
    wid = c * 16 + s

    def fill(i, _):
        for k in range(DEGW // 16):
            z_v[i, pl.ds(k * 16, 16)] = jnp.zeros((16,), jnp.float32)
            one_v[i, pl.ds(k * 16, 16)] = jnp.ones((16,), jnp.float32)
        return 0

    lax.fori_loop(0, CHUNK, fill, 0)
    for k in range(ROWS_PER_TILE // CHUNK):
        off = s * ROWS_PER_TILE + k * CHUNK
        pltpu.sync_copy(z_v, acc0.at[pl.ds(off, CHUNK)])
        pltpu.sync_copy(z_v, acc1.at[pl.ds(off, CHUNK)])
    plsc.subcore_barrier()

    pltpu.sync_copy(src_hbm.at[pl.ds(wid * CHUNKS_PER_TILE, CHUNKS_PER_TILE)], src_v)
    pltpu.sync_copy(dst_hbm.at[pl.ds(wid * CHUNKS_PER_TILE, CHUNKS_PER_TILE)], dst_v)

    def body(j, _):
        pltpu.sync_copy(one_v, acc0.at[dst_v.at[j]], add=True)
        pltpu.sync_copy(one_v, acc1.at[src_v.at[j]], add=True)
        return 0

    lax.fori_loop(0, CHUNKS_PER_TILE, body, 0)
    plsc.subcore_barrier()
    pltpu.sync_copy(acc0.at[pl.ds(s * ROWS_PER_TILE, ROWS_PER_TILE)],
                    deg0_out.at[c, pl.ds(s * ROWS_PER_TILE, ROWS_PER_TILE)])
    pltpu.sync_copy(acc1.at[pl.ds(s * ROWS_PER_TILE, ROWS_PER_TILE)],
                    deg1_out.at[c, pl.ds(s * ROWS_PER_TILE, ROWS_PER_TILE)])


@functools.partial(
    pl.kernel,
    out_type=[jax.ShapeDtypeStruct((2, NPAD, HID), jnp.float32),
              jax.ShapeDtypeStruct((2, NPAD, HID), jnp.float32)],
    mesh=_MESH,
    scratch_types=(
        [pltpu.VMEM((PBLK, PCHUNK), jnp.int32),
         pltpu.VMEM((PBLK, PCHUNK), jnp.int32)]
        + [pltpu.VMEM((PCHUNK, HID), jnp.float32) for _ in range(2 * NBUF)]
        + [pltpu.SemaphoreType.DMA for _ in range(2 * NBUF + 2)]
        + [pltpu.VMEM_SHARED((NPAD, HID), jnp.float32),
           pltpu.VMEM_SHARED((NPAD, HID), jnp.float32)]
    ),
    compiler_params=pltpu.CompilerParams(use_tc_tiling_on_sc=False),
)
def _sc_propagate(u0_hbm, u1_hbm, src_hbm, dst_hbm, s0_out, s1_out,
                  src_v, dst_v, *rest):
    r0 = rest[0:NBUF]
    r1 = rest[NBUF:2 * NBUF]
    g0 = rest[2 * NBUF:3 * NBUF]
    g1 = rest[3 * NBUF:4 * NBUF]
    s0sem = rest[4 * NBUF]
    s1sem = rest[4 * NBUF + 1]
    acc0 = rest[4 * NBUF + 2]
    acc1 = rest[4 * NBUF + 3]
    c = lax.axis_index("c")
    s = lax.axis_index("s")

    def fill(i, _):
        for k in range(HID // 16):
            r0[0][i, pl.ds(k * 16, 16)] = jnp.zeros((16,), jnp.float32)
        return 0

    lax.fori_loop(0, PCHUNK, fill, 0)
    for k in range(ROWS_PER_TILE // PCHUNK):
        off = s * ROWS_PER_TILE + k * PCHUNK
        pltpu.async_copy(r0[0], acc0.at[pl.ds(off, PCHUNK)], g0[0])
        pltpu.async_copy(r0[0], acc1.at[pl.ds(off, PCHUNK)], g0[0])
    for k in range(ROWS_PER_TILE // PCHUNK):
        off = s * ROWS_PER_TILE + k * PCHUNK
        pltpu.make_async_copy(r0[0], acc0.at[pl.ds(off, PCHUNK)], g0[0]).wait()
        pltpu.make_async_copy(r0[0], acc1.at[pl.ds(off, PCHUNK)], g0[0]).wait()
    plsc.subcore_barrier()

    def run(tile_base, n_chunks):
        for ib in range(n_chunks // PBLK):
            base = tile_base + ib * PBLK
            pltpu.sync_copy(src_hbm.at[pl.ds(base, PBLK)], src_v)
            pltpu.sync_copy(dst_hbm.at[pl.ds(base, PBLK)], dst_v)

            for p in range(NBUF - 1):
                pltpu.async_copy(u0_hbm.at[src_v.at[p]], r0[p], g0[p])
                pltpu.async_copy(u1_hbm.at[dst_v.at[p]], r1[p], g1[p])

            def block(bi, _):
                for t in range(NBUF):
                    j = bi * NBUF + t
                    nf = (t + NBUF - 1) % NBUF

                    @pl.when(j >= 1)
                    def _():
                        pltpu.make_async_copy(
                            r0[nf], acc0.at[dst_v.at[j - 1]], s0sem).wait()
                        pltpu.make_async_copy(
                            r1[nf], acc1.at[src_v.at[j - 1]], s1sem).wait()

                    @pl.when(j + NBUF - 1 < PBLK)
                    def _():
                        pltpu.async_copy(
                            u0_hbm.at[src_v.at[j + NBUF - 1]], r0[nf], g0[nf])
                        pltpu.async_copy(
                            u1_hbm.at[dst_v.at[j + NBUF - 1]], r1[nf], g1[nf])

                    pltpu.make_async_copy(u0_hbm.at[src_v.at[j]], r0[t], g0[t]).wait()
                    pltpu.async_copy(r0[t], acc0.at[dst_v.at[j]], s0sem, add=True)
                    pltpu.make_async_copy(u1_hbm.at[dst_v.at[j]], r1[t], g1[t]).wait()
                    pltpu.async_copy(r1[t], acc1.at[src_v.at[j]], s1sem, add=True)
                return 0

            lax.fori_loop(0, PBLK // NBUF, block, 0)
            last = PBLK - 1
            lt = last % NBUF
            pltpu.make_async_copy(r0[lt], acc0.at[dst_v.at[last]], s0sem).wait()
            pltpu.make_async_copy(r1[lt], acc1.at[src_v.at[last]], s1sem).wait()

    @pl.when(c == 0)
    def _():
        run(s * CHUNKS_C0, CHUNKS_C0)

    @pl.when(c == 1)
    def _():
        run(16 * CHUNKS_C0 + s * CHUNKS_C1, CHUNKS_C1)

    plsc.subcore_barrier()
    pltpu.async_copy(acc0.at[pl.ds(s * ROWS_PER_TILE, ROWS_PER_TILE)],
                     s0_out.at[c, pl.ds(s * ROWS_PER_TILE, ROWS_PER_TILE)], g0[0])
    pltpu.async_copy(acc1.at[pl.ds(s * ROWS_PER_TILE, ROWS_PER_TILE)],
                     s1_out.at[c, pl.ds(s * ROWS_PER_TILE, ROWS_PER_TILE)], g0[1])
    pltpu.make_async_copy(acc0.at[pl.ds(s * ROWS_PER_TILE, ROWS_PER_TILE)],
                          s0_out.at[c, pl.ds(s * ROWS_PER_TILE, ROWS_PER_TILE)], g0[0]).wait()
    pltpu.make_async_copy(acc1.at[pl.ds(s * ROWS_PER_TILE, ROWS_PER_TILE)],
                          s1_out.at[c, pl.ds(s * ROWS_PER_TILE, ROWS_PER_TILE)], g0[1]).wait()


_BLK = 256
_GRID = NPAD // _BLK


def _tc_prep_body(x_ref, d0_ref, d1_ref, w00_ref, w10_ref,
                  u0_ref, u1_ref, di0_ref, di1_ref):
    deg0 = d0_ref[0, :, 0] + d0_ref[1, :, 0] + 1.0
    deg1 = d1_ref[0, :, 0] + d1_ref[1, :, 0] + 1.0
    dinv0 = lax.rsqrt(deg0)
    dinv1 = lax.rsqrt(deg1)
    x = x_ref[...]
    u0_ref[...] = jnp.dot(x, w00_ref[...], preferred_element_type=jnp.float32) * dinv0[:, None]
    u1_ref[...] = jnp.dot(x, w10_ref[...], preferred_element_type=jnp.float32) * dinv1[:, None]
    di0_ref[...] = dinv0
    di1_ref[...] = dinv1


def _tc_prep(x_pad, deg0p, deg1p, w00, w10):
    return pl.pallas_call(
        _tc_prep_body,
        grid=(_GRID,),
        in_specs=[
            pl.BlockSpec((_BLK, IN_CH), lambda i: (i, 0)),
            pl.BlockSpec((2, _BLK, DEGW), lambda i: (0, i, 0)),
            pl.BlockSpec((2, _BLK, DEGW), lambda i: (0, i, 0)),
            pl.BlockSpec((IN_CH, HID), lambda i: (0, 0)),
            pl.BlockSpec((IN_CH, HID), lambda i: (0, 0)),
        ],
        out_specs=[
            pl.BlockSpec((_BLK, HID), lambda i: (i, 0)),
            pl.BlockSpec((_BLK, HID), lambda i: (i, 0)),
            pl.BlockSpec((_BLK,), lambda i: (i,)),
            pl.BlockSpec((_BLK,), lambda i: (i,)),
        ],
        out_shape=[
            jax.ShapeDtypeStruct((NPAD, HID), jnp.float32),
            jax.ShapeDtypeStruct((NPAD, HID), jnp.float32),
            jax.ShapeDtypeStruct((NPAD,), jnp.float32),
            jax.ShapeDtypeStruct((NPAD,), jnp.float32),
        ],
    )(x_pad, deg0p, deg1p, w00, w10)


def _tc_mid_body(s0_ref, s1_ref, u0_ref, u1_ref, di0_ref, di1_ref,
                 b00_ref, b10_ref, w01_ref, w11_ref, v0_ref, v1_ref):
    di0 = di0_ref[...][:, None]
    di1 = di1_ref[...][:, None]
    h0 = jnp.maximum(di0 * (s0_ref[0] + s0_ref[1] + u0_ref[...]) + b00_ref[...], 0.0)
    h1 = jnp.maximum(di1 * (s1_ref[0] + s1_ref[1] + u1_ref[...]) + b10_ref[...], 0.0)
    v0_ref[...] = jnp.dot(h0, w01_ref[...], preferred_element_type=jnp.float32) * di0
    v1_ref[...] = jnp.dot(h1, w11_ref[...], preferred_element_type=jnp.float32) * di1


def _tc_mid(s0p, s1p, u0, u1, di0, di1, b00, b10, w01, w11):
    return pl.pallas_call(
        _tc_mid_body,
        grid=(_GRID,),
        in_specs=[
            pl.BlockSpec((2, _BLK, HID), lambda i: (0, i, 0)),
            pl.BlockSpec((2, _BLK, HID), lambda i: (0, i, 0)),
            pl.BlockSpec((_BLK, HID), lambda i: (i, 0)),
            pl.BlockSpec((_BLK, HID), lambda i: (i, 0)),
            pl.BlockSpec((_BLK,), lambda i: (i,)),
            pl.BlockSpec((_BLK,), lambda i: (i,)),
            pl.BlockSpec((HID,), lambda i: (0,)),
            pl.BlockSpec((HID,), lambda i: (0,)),
            pl.BlockSpec((HID, HID), lambda i: (0, 0)),
            pl.BlockSpec((HID, HID), lambda i: (0, 0)),
        ],
        out_specs=[
            pl.BlockSpec((_BLK, HID), lambda i: (i, 0)),
            pl.BlockSpec((_BLK, HID), lambda i: (i, 0)),
        ],
        out_shape=[
            jax.ShapeDtypeStruct((NPAD, HID), jnp.float32),
            jax.ShapeDtypeStruct((NPAD, HID), jnp.float32),
        ],
    )(s0p, s1p, u0, u1, di0, di1, b00, b10, w01, w11)


def _tc_final_body(s0_ref, s1_ref, v0_ref, v1_ref, di0_ref, di1_ref,
                   b01_ref, b11_ref, out_ref):
    di0 = di0_ref[...][:, None]
    di1 = di1_ref[...][:, None]
    out_ref[:, :HID] = di0 * (s0_ref[0] + s0_ref[1] + v0_ref[...]) + b01_ref[...]
    out_ref[:, HID:] = di1 * (s1_ref[0] + s1_ref[1] + v1_ref[...]) + b11_ref[...]


def _tc_final(s0p, s1p, v0, v1, di0, di1, b01, b11):
    return pl.pallas_call(
        _tc_final_body,
        grid=(_GRID,),
        in_specs=[
            pl.BlockSpec((2, _BLK, HID), lambda i: (0, i, 0)),
            pl.BlockSpec((2, _BLK, HID), lambda i: (0, i, 0)),
            pl.BlockSpec((_BLK, HID), lambda i: (i, 0)),
            pl.BlockSpec((_BLK, HID), lambda i: (i, 0)),
            pl.BlockSpec((_BLK,), lambda i: (i,)),
            pl.BlockSpec((_BLK,), lambda i: (i,)),
            pl.BlockSpec((HID,), lambda i: (0,)),
            pl.BlockSpec((HID,), lambda i: (0,)),
        ],
        out_specs=pl.BlockSpec((_BLK, 2 * HID), lambda i: (i, 0)),
        out_shape=jax.ShapeDtypeStruct((NPAD, 2 * HID), jnp.float32),
    )(s0p, s1p, v0, v1, di0, di1, b01, b11)


def kernel(x, edge_index, adj_mat_ls, W0_0, b0_0, W0_1, b0_1, W1_0, b1_0, W1_1, b1_1):
    ei = edge_index.astype(jnp.int32)
    pad = jnp.full((EPAD - N_EDGES,), N_NODES, jnp.int32)
    src2d = jnp.concatenate([ei[0], pad]).reshape(EPAD // CHUNK, CHUNK)
    dst2d = jnp.concatenate([ei[1], pad]).reshape(EPAD // CHUNK, CHUNK)
    x_pad = jnp.pad(x, ((0, NPAD - N_NODES), (0, 0)))

    src64 = src2d.reshape(EPAD // PCHUNK, PCHUNK)
    dst64 = dst2d.reshape(EPAD // PCHUNK, PCHUNK)

    deg0p, deg1p = _sc_degrees(src2d, dst2d)
    u0, u1, di0, di1 = _tc_prep(x_pad, deg0p, deg1p, W0_0, W1_0)
    s0p, s1p = _sc_propagate(u0, u1, src64, dst64)
    v0, v1 = _tc_mid(s0p, s1p, u0, u1, di0, di1, b0_0, b1_0, W0_1, W1_1)
    t0p, t1p = _sc_propagate(v0, v1, src64, dst64)
    out = _tc_final(t0p, t1p, v0, v1, di0, di1, b0_1, b1_1)
    return out[:N_NODES]
